# Initial kernel scaffold; baseline (speedup 1.0000x reference)
import jax
import jax.numpy as jnp
from jax.experimental import pallas as pl


def kernel(predicts, similarities, adjList):
    raise NotImplementedError("write your pallas kernel here")


# speedup vs baseline: 2.1832x; 2.1832x over previous
import jax
import jax.numpy as jnp
from jax.experimental import pallas as pl
from jax.experimental.pallas import tpu as pltpu


def _loss_body(p_ref, s_ref, a_ref, out_ref):
    p = p_ref[...]
    s = s_ref[...]
    inv_l = jnp.float32(1.0 / p.shape[1])
    cand = jnp.dot(p, a_ref[...], preferred_element_type=jnp.float32) * inv_l
    diff = p - jnp.dot(s, cand, preferred_element_type=jnp.float32)
    sq = jnp.sum(diff * diff, axis=1)
    norms = jnp.sqrt(sq)
    row_sums = jnp.sum(s, axis=1)
    mask = row_sums != 0
    cnt = jnp.sum(mask.astype(jnp.float32))
    total = jnp.sum(jnp.where(mask, norms, jnp.float32(0.0)))
    out_ref[...] = jnp.reshape(total / cnt, (1, 1))


def kernel(predicts, similarities, adjList):
    out = pl.pallas_call(
        _loss_body,
        out_shape=jax.ShapeDtypeStruct((1, 1), jnp.float32),
    )(predicts, similarities, adjList)
    return out[0, 0]
